# baseline (speedup 1.0000x reference)
import functools

import jax
import jax.numpy as jnp
from jax import lax
from jax.experimental import pallas as pl
from jax.experimental.pallas import tpu as pltpu
from jax.experimental.pallas import tpu_sc as plsc

N_NODES = 10000
N_EDGES = 320000
IN_FEATS = 128
H_FEATS = 256
NUM_CLASSES = 64

NC, NS = 2, 16
NW = NC * NS
K = 80
ZR = 208
ROWS_PER_TILE = 624
TAIL_BASE = NS * ROWS_PER_TILE
TAIL_ROWS = N_NODES - TAIL_BASE
DEG_CHUNK = 1000


def _make_seg_sum(d, with_deg):
    e_per_w = N_EDGES // NW
    n_chunks = e_per_w // K
    assert N_EDGES % NW == 0 and e_per_w % K == 0
    n_pairs = n_chunks // 2
    tail_chunks = n_chunks - 2 * n_pairs

    out_types = [jax.ShapeDtypeStruct((NC * N_NODES, d), jnp.float32)]
    scratch = [
        pltpu.VMEM((e_per_w,), jnp.int32),
        pltpu.VMEM((n_chunks, K), jnp.int32),
        pltpu.VMEM((K, d), jnp.float32),
        pltpu.VMEM((K, d), jnp.float32),
        pltpu.VMEM_SHARED((N_NODES, d), jnp.float32),
        pltpu.SemaphoreType.DMA,
        pltpu.SemaphoreType.DMA,
        pltpu.SemaphoreType.DMA,
        pltpu.SemaphoreType.DMA,
    ]
    if with_deg:
        out_types.append(jax.ShapeDtypeStruct((NC * N_NODES,), jnp.float32))
        scratch += [
            pltpu.VMEM((K,), jnp.float32),
            pltpu.VMEM((1008,), jnp.float32),
            pltpu.VMEM_SHARED((N_NODES,), jnp.float32),
            pltpu.SemaphoreType.DMA,
        ]

    mesh = plsc.VectorSubcoreMesh(
        core_axis_name="c", subcore_axis_name="s",
        num_cores=NC, num_subcores=NS)

    @functools.partial(pl.kernel, out_type=out_types, mesh=mesh,
                       scratch_types=scratch)
    def seg_sum(feats, src, dstr, *refs):
        if with_deg:
            (out_hbm, deg_hbm, sidx, didx, rows0, rows1, acc,
             gsem0, gsem1, ssem0, ssem1, ones, zdeg, dacc, degsem) = refs
        else:
            (out_hbm, sidx, didx, rows0, rows1, acc,
             gsem0, gsem1, ssem0, ssem1) = refs
        c = lax.axis_index("c")
        s = lax.axis_index("s")
        wid = s * NC + c
        rows = (rows0, rows1)
        gsem = (gsem0, gsem1)
        ssem = (ssem0, ssem1)

        pltpu.sync_copy(src.at[pl.ds(wid * e_per_w, e_per_w)], sidx)
        pltpu.sync_copy(dstr.at[wid], didx)

        def zrow_body(i, carry):
            for j in range(d // 16):
                rows0[i, pl.ds(j * 16, 16)] = jnp.zeros((16,), jnp.float32)
            return carry
        lax.fori_loop(0, K, zrow_body, 0)
        if with_deg:
            for j in range(K // 16):
                ones[pl.ds(j * 16, 16)] = jnp.ones((16,), jnp.float32)

            def zdeg_body(i, carry):
                zdeg[pl.ds(i * 16, 16)] = jnp.zeros((16,), jnp.float32)
                return carry
            lax.fori_loop(0, 1008 // 16, zdeg_body, 0)

        for z in range(ROWS_PER_TILE // K):
            pltpu.sync_copy(
                rows0, acc.at[pl.ds(s * ROWS_PER_TILE + z * K, K)])
        z_rem = ROWS_PER_TILE - (ROWS_PER_TILE // K) * K
        if z_rem:
            pltpu.sync_copy(
                rows0.at[pl.ds(0, z_rem)],
                acc.at[pl.ds(s * ROWS_PER_TILE + ROWS_PER_TILE - z_rem,
                             z_rem)])
        @pl.when(s == 0)
        def _():
            pltpu.sync_copy(rows0.at[pl.ds(0, TAIL_ROWS)],
                            acc.at[pl.ds(TAIL_BASE, TAIL_ROWS)])
        if with_deg:
            @pl.when(s < N_NODES // DEG_CHUNK)
            def _():
                pltpu.sync_copy(zdeg.at[pl.ds(0, DEG_CHUNK)],
                                dacc.at[pl.ds(s * DEG_CHUNK, DEG_CHUNK)])
        plsc.subcore_barrier()

        h = K // 2

        def gather(k, b):
            pltpu.async_copy(
                feats.at[sidx.at[pl.ds(k * K, h)]],
                rows[b].at[pl.ds(0, h)], gsem[b])
            return pltpu.async_copy(
                feats.at[sidx.at[pl.ds(k * K + h, h)]],
                rows[b].at[pl.ds(h, h)], gsem[b])

        def scatter(k, b):
            pltpu.async_copy(rows[b], acc.at[didx.at[k]], ssem[b], add=True)
            if with_deg:
                pltpu.async_copy(ones, dacc.at[didx.at[k]], degsem, add=True)

        def wait_scatter(b):
            pltpu.make_async_copy(rows[b], acc.at[didx.at[0]], ssem[b]).wait()

        def wait_gather(b):
            pltpu.make_async_copy(
                feats.at[sidx.at[pl.ds(0, K)]], rows[b], gsem[b]).wait()

        gather(0, 0)
        gather(1, 1)
        wait_gather(0)
        scatter(0, 0)
        wait_gather(1)
        scatter(1, 1)

        def pair(k2, carry):
            c0 = 2 * k2
            for b in range(2):
                wait_scatter(b)
                gather(c0 + b, b)
            for b in range(2):
                wait_gather(b)
                scatter(c0 + b, b)
            return carry
        lax.fori_loop(1, n_pairs, pair, 0)

        for t in range(tail_chunks):
            k = 2 * n_pairs + t
            wait_scatter(0)
            gather(k, 0)
            wait_gather(0)
            scatter(k, 0)

        for b in range(2):
            wait_scatter(b)
        if with_deg:
            def deg_drain(i, carry):
                pltpu.make_async_copy(
                    ones, dacc.at[didx.at[0]], degsem).wait()
                return carry
            lax.fori_loop(0, n_chunks, deg_drain, 0)
        plsc.subcore_barrier()

        pltpu.sync_copy(
            acc.at[pl.ds(s * ROWS_PER_TILE, ROWS_PER_TILE)],
            out_hbm.at[pl.ds(c * N_NODES + s * ROWS_PER_TILE, ROWS_PER_TILE)])
        @pl.when(s == 0)
        def _():
            pltpu.sync_copy(
                acc.at[pl.ds(TAIL_BASE, TAIL_ROWS)],
                out_hbm.at[pl.ds(c * N_NODES + TAIL_BASE, TAIL_ROWS)])
        if with_deg:
            @pl.when(s < N_NODES // DEG_CHUNK)
            def _():
                pltpu.sync_copy(dacc.at[pl.ds(s * DEG_CHUNK, DEG_CHUNK)],
                                zdeg.at[pl.ds(0, DEG_CHUNK)])
                pltpu.sync_copy(
                    zdeg.at[pl.ds(0, DEG_CHUNK)],
                    deg_hbm.at[pl.ds(c * N_NODES + s * DEG_CHUNK, DEG_CHUNK)])

    return seg_sum


_seg_sum_x = _make_seg_sum(IN_FEATS, with_deg=True)
_seg_sum_g = _make_seg_sum(IN_FEATS, with_deg=False)

_NB = 1000


def _dense_body(p0, p1, d0, d1, w1t, b1, w2t, g):
    deg = jnp.maximum(d0[...] + d1[...], 1.0)
    a = (p0[...] + p1[...]) / deg
    h = jnp.dot(a, w1t[...], preferred_element_type=jnp.float32) + b1[...]
    h = jnp.maximum(h, 0.0)
    g2 = jnp.dot(h, w2t[...], preferred_element_type=jnp.float32)
    g[...] = jnp.concatenate(
        [g2, jnp.zeros((_NB, IN_FEATS - NUM_CLASSES), jnp.float32)], axis=1)


def _combine_body(q0, q1, d0, d1, b2, o):
    deg = jnp.maximum(d0[...] + d1[...], 1.0)
    q = q0[:, :NUM_CLASSES] + q1[:, :NUM_CLASSES]
    o[...] = q / deg + b2[...]


_dense = pl.pallas_call(
    _dense_body,
    grid=(N_NODES // _NB,),
    in_specs=[
        pl.BlockSpec((_NB, IN_FEATS), lambda i: (i, 0)),
        pl.BlockSpec((_NB, IN_FEATS), lambda i: (i, 0)),
        pl.BlockSpec((_NB, 1), lambda i: (i, 0)),
        pl.BlockSpec((_NB, 1), lambda i: (i, 0)),
        pl.BlockSpec((IN_FEATS, H_FEATS), lambda i: (0, 0)),
        pl.BlockSpec((1, H_FEATS), lambda i: (0, 0)),
        pl.BlockSpec((H_FEATS, NUM_CLASSES), lambda i: (0, 0)),
    ],
    out_specs=pl.BlockSpec((_NB, IN_FEATS), lambda i: (i, 0)),
    out_shape=jax.ShapeDtypeStruct((N_NODES, IN_FEATS), jnp.float32),
)

_combine = pl.pallas_call(
    _combine_body,
    grid=(N_NODES // _NB,),
    in_specs=[
        pl.BlockSpec((_NB, IN_FEATS), lambda i: (i, 0)),
        pl.BlockSpec((_NB, IN_FEATS), lambda i: (i, 0)),
        pl.BlockSpec((_NB, 1), lambda i: (i, 0)),
        pl.BlockSpec((_NB, 1), lambda i: (i, 0)),
        pl.BlockSpec((1, NUM_CLASSES), lambda i: (0, 0)),
    ],
    out_specs=pl.BlockSpec((_NB, NUM_CLASSES), lambda i: (i, 0)),
    out_shape=jax.ShapeDtypeStruct((N_NODES, NUM_CLASSES), jnp.float32),
)


_N_CHUNKS = N_EDGES // NW // K


def kernel(x, edge_index, W1, b1, W2, b2):
    src = edge_index[0].astype(jnp.int32)
    dst = edge_index[1].astype(jnp.int32).reshape(NW, _N_CHUNKS, K)

    part1, degp = _seg_sum_x(x, src, dst)
    d0 = degp[:N_NODES].reshape(N_NODES, 1)
    d1 = degp[N_NODES:].reshape(N_NODES, 1)
    g = _dense(part1[:N_NODES], part1[N_NODES:], d0, d1,
               W1.T, b1.reshape(1, H_FEATS), W2.T)
    (part2,) = _seg_sum_g(g, src, dst)
    out = _combine(part2[:N_NODES], part2[N_NODES:], d0, d1,
                   b2.reshape(1, NUM_CLASSES))
    return out


# speedup vs baseline: 1.0438x; 1.0438x over previous
import functools

import jax
import jax.numpy as jnp
from jax import lax
from jax.experimental import pallas as pl
from jax.experimental.pallas import tpu as pltpu
from jax.experimental.pallas import tpu_sc as plsc

N_NODES = 10000
N_EDGES = 320000
IN_FEATS = 128
H_FEATS = 256
NUM_CLASSES = 64

NC, NS = 2, 16
NW = NC * NS
K = 80
ZR = 208
ROWS_PER_TILE = 624
TAIL_BASE = NS * ROWS_PER_TILE
TAIL_ROWS = N_NODES - TAIL_BASE
DEG_CHUNK = 1000


def _make_seg_sum(d, with_deg):
    e_per_w = N_EDGES // NW
    n_chunks = e_per_w // K
    assert N_EDGES % NW == 0 and e_per_w % K == 0
    n_pairs = n_chunks // 2
    tail_chunks = n_chunks - 2 * n_pairs

    out_types = [jax.ShapeDtypeStruct((NC * N_NODES, d), jnp.float32)]
    scratch = [
        pltpu.VMEM((e_per_w,), jnp.int32),
        pltpu.VMEM((n_chunks, K), jnp.int32),
        pltpu.VMEM((K, d), jnp.float32),
        pltpu.VMEM((K, d), jnp.float32),
        pltpu.VMEM_SHARED((N_NODES, d), jnp.float32),
        pltpu.SemaphoreType.DMA,
        pltpu.SemaphoreType.DMA,
        pltpu.SemaphoreType.DMA,
        pltpu.SemaphoreType.DMA,
    ]
    if with_deg:
        out_types.append(jax.ShapeDtypeStruct((NC * N_NODES,), jnp.float32))
        scratch += [
            pltpu.VMEM((K,), jnp.float32),
            pltpu.VMEM((1008,), jnp.float32),
            pltpu.VMEM_SHARED((N_NODES,), jnp.float32),
            pltpu.SemaphoreType.DMA,
        ]

    mesh = plsc.VectorSubcoreMesh(
        core_axis_name="c", subcore_axis_name="s",
        num_cores=NC, num_subcores=NS)

    @functools.partial(pl.kernel, out_type=out_types, mesh=mesh,
                       scratch_types=scratch)
    def seg_sum(feats, src, dstr, *refs):
        if with_deg:
            (out_hbm, deg_hbm, sidx, didx, rows0, rows1, acc,
             gsem0, gsem1, ssem0, ssem1, ones, zdeg, dacc, degsem) = refs
        else:
            (out_hbm, sidx, didx, rows0, rows1, acc,
             gsem0, gsem1, ssem0, ssem1) = refs
        c = lax.axis_index("c")
        s = lax.axis_index("s")
        wid = s * NC + c
        rows = (rows0, rows1)
        gsem = (gsem0, gsem1)
        ssem = (ssem0, ssem1)

        pltpu.sync_copy(src.at[pl.ds(wid * e_per_w, e_per_w)], sidx)
        pltpu.sync_copy(dstr.at[wid], didx)

        def zrow_body(i, carry):
            for j in range(d // 16):
                rows0[i, pl.ds(j * 16, 16)] = jnp.zeros((16,), jnp.float32)
            return carry
        lax.fori_loop(0, K, zrow_body, 0)
        if with_deg:
            for j in range(K // 16):
                ones[pl.ds(j * 16, 16)] = jnp.ones((16,), jnp.float32)

            def zdeg_body(i, carry):
                zdeg[pl.ds(i * 16, 16)] = jnp.zeros((16,), jnp.float32)
                return carry
            lax.fori_loop(0, 1008 // 16, zdeg_body, 0)

        for z in range(ROWS_PER_TILE // K):
            pltpu.sync_copy(
                rows0, acc.at[pl.ds(s * ROWS_PER_TILE + z * K, K)])
        z_rem = ROWS_PER_TILE - (ROWS_PER_TILE // K) * K
        if z_rem:
            pltpu.sync_copy(
                rows0.at[pl.ds(0, z_rem)],
                acc.at[pl.ds(s * ROWS_PER_TILE + ROWS_PER_TILE - z_rem,
                             z_rem)])
        @pl.when(s == 0)
        def _():
            pltpu.sync_copy(rows0.at[pl.ds(0, TAIL_ROWS)],
                            acc.at[pl.ds(TAIL_BASE, TAIL_ROWS)])
        if with_deg:
            @pl.when(s < N_NODES // DEG_CHUNK)
            def _():
                pltpu.sync_copy(zdeg.at[pl.ds(0, DEG_CHUNK)],
                                dacc.at[pl.ds(s * DEG_CHUNK, DEG_CHUNK)])
        plsc.subcore_barrier()

        def gather(k, b):
            pltpu.async_copy(
                feats.at[sidx.at[pl.ds(k * K, K)]], rows[b], gsem[b])

        def scatter(k, b):
            pltpu.async_copy(rows[b], acc.at[didx.at[k]], ssem[b], add=True)
            if with_deg:
                pltpu.async_copy(ones, dacc.at[didx.at[k]], degsem, add=True)

        def wait_scatter(b):
            pltpu.make_async_copy(rows[b], acc.at[didx.at[0]], ssem[b]).wait()

        def wait_gather(b):
            pltpu.make_async_copy(
                feats.at[sidx.at[pl.ds(0, K)]], rows[b], gsem[b]).wait()

        gather(0, 0)
        gather(1, 1)
        wait_gather(0)
        scatter(0, 0)
        wait_gather(1)
        scatter(1, 1)

        def pair(k2, carry):
            c0 = 2 * k2
            for b in range(2):
                wait_scatter(b)
                gather(c0 + b, b)
            for b in range(2):
                wait_gather(b)
                scatter(c0 + b, b)
            return carry
        lax.fori_loop(1, n_pairs, pair, 0)

        for t in range(tail_chunks):
            k = 2 * n_pairs + t
            wait_scatter(0)
            gather(k, 0)
            wait_gather(0)
            scatter(k, 0)

        for b in range(2):
            wait_scatter(b)
        if with_deg:
            def deg_drain(i, carry):
                pltpu.make_async_copy(
                    ones, dacc.at[didx.at[0]], degsem).wait()
                return carry
            lax.fori_loop(0, n_chunks, deg_drain, 0)
        plsc.subcore_barrier()

        pltpu.sync_copy(
            acc.at[pl.ds(s * ROWS_PER_TILE, ROWS_PER_TILE)],
            out_hbm.at[pl.ds(c * N_NODES + s * ROWS_PER_TILE, ROWS_PER_TILE)])
        @pl.when(s == 0)
        def _():
            pltpu.sync_copy(
                acc.at[pl.ds(TAIL_BASE, TAIL_ROWS)],
                out_hbm.at[pl.ds(c * N_NODES + TAIL_BASE, TAIL_ROWS)])
        if with_deg:
            @pl.when(s < N_NODES // DEG_CHUNK)
            def _():
                pltpu.sync_copy(dacc.at[pl.ds(s * DEG_CHUNK, DEG_CHUNK)],
                                zdeg.at[pl.ds(0, DEG_CHUNK)])
                pltpu.sync_copy(
                    zdeg.at[pl.ds(0, DEG_CHUNK)],
                    deg_hbm.at[pl.ds(c * N_NODES + s * DEG_CHUNK, DEG_CHUNK)])

    return seg_sum


_seg_sum_x = _make_seg_sum(IN_FEATS, with_deg=True)
_seg_sum_g = _make_seg_sum(IN_FEATS, with_deg=False)

_NB = 1000


def _dense_body(p0, p1, d0, d1, w1t, b1, w2t, g):
    deg = jnp.maximum(d0[...] + d1[...], 1.0)
    a = (p0[...] + p1[...]) / deg
    h = jnp.dot(a, w1t[...], preferred_element_type=jnp.float32) + b1[...]
    h = jnp.maximum(h, 0.0)
    g2 = jnp.dot(h, w2t[...], preferred_element_type=jnp.float32)
    g[...] = jnp.concatenate(
        [g2, jnp.zeros((_NB, IN_FEATS - NUM_CLASSES), jnp.float32)], axis=1)


def _combine_body(q0, q1, d0, d1, b2, o):
    deg = jnp.maximum(d0[...] + d1[...], 1.0)
    q = q0[:, :NUM_CLASSES] + q1[:, :NUM_CLASSES]
    o[...] = q / deg + b2[...]


_NBLK = N_NODES // _NB

_dense = pl.pallas_call(
    _dense_body,
    grid=(_NBLK,),
    in_specs=[
        pl.BlockSpec((_NB, IN_FEATS), lambda i: (i, 0)),
        pl.BlockSpec((_NB, IN_FEATS), lambda i: (i + _NBLK, 0)),
        pl.BlockSpec((_NB, 1), lambda i: (i, 0)),
        pl.BlockSpec((_NB, 1), lambda i: (i + _NBLK, 0)),
        pl.BlockSpec((IN_FEATS, H_FEATS), lambda i: (0, 0)),
        pl.BlockSpec((1, H_FEATS), lambda i: (0, 0)),
        pl.BlockSpec((H_FEATS, NUM_CLASSES), lambda i: (0, 0)),
    ],
    out_specs=pl.BlockSpec((_NB, IN_FEATS), lambda i: (i, 0)),
    out_shape=jax.ShapeDtypeStruct((N_NODES, IN_FEATS), jnp.float32),
)

_combine = pl.pallas_call(
    _combine_body,
    grid=(_NBLK,),
    in_specs=[
        pl.BlockSpec((_NB, IN_FEATS), lambda i: (i, 0)),
        pl.BlockSpec((_NB, IN_FEATS), lambda i: (i + _NBLK, 0)),
        pl.BlockSpec((_NB, 1), lambda i: (i, 0)),
        pl.BlockSpec((_NB, 1), lambda i: (i + _NBLK, 0)),
        pl.BlockSpec((1, NUM_CLASSES), lambda i: (0, 0)),
    ],
    out_specs=pl.BlockSpec((_NB, NUM_CLASSES), lambda i: (i, 0)),
    out_shape=jax.ShapeDtypeStruct((N_NODES, NUM_CLASSES), jnp.float32),
)


_N_CHUNKS = N_EDGES // NW // K


def kernel(x, edge_index, W1, b1, W2, b2):
    src = edge_index[0].astype(jnp.int32)
    dst = edge_index[1].astype(jnp.int32).reshape(NW, _N_CHUNKS, K)

    part1, degp = _seg_sum_x(x, src, dst)
    degc = degp.reshape(NC * N_NODES, 1)
    g = _dense(part1, part1, degc, degc,
               W1.T, b1.reshape(1, H_FEATS), W2.T)
    (part2,) = _seg_sum_g(g, src, dst)
    out = _combine(part2, part2, degc, degc,
                   b2.reshape(1, NUM_CLASSES))
    return out


# speedup vs baseline: 1.0617x; 1.0172x over previous
import functools

import jax
import jax.numpy as jnp
from jax import lax
from jax.experimental import pallas as pl
from jax.experimental.pallas import tpu as pltpu
from jax.experimental.pallas import tpu_sc as plsc

N_NODES = 10000
N_EDGES = 320000
IN_FEATS = 128
H_FEATS = 256
NUM_CLASSES = 64

NC, NS = 2, 16
NW = NC * NS
K = 80
ZR = 208
ROWS_PER_TILE = 624
TAIL_BASE = NS * ROWS_PER_TILE
TAIL_ROWS = N_NODES - TAIL_BASE
DEG_CHUNK = 1000


def _make_seg_sum(d, with_deg):
    e_per_w = N_EDGES // NW
    n_chunks = e_per_w // K
    assert N_EDGES % NW == 0 and e_per_w % K == 0
    n_pairs = n_chunks // 2
    tail_chunks = n_chunks - 2 * n_pairs

    out_types = [jax.ShapeDtypeStruct((NC * N_NODES, d), jnp.float32)]
    scratch = [
        pltpu.VMEM((e_per_w,), jnp.int32),
        pltpu.VMEM((n_chunks, K), jnp.int32),
        pltpu.VMEM((K, d), jnp.float32),
        pltpu.VMEM((K, d), jnp.float32),
        pltpu.VMEM_SHARED((N_NODES, d), jnp.float32),
        pltpu.SemaphoreType.DMA,
        pltpu.SemaphoreType.DMA,
        pltpu.SemaphoreType.DMA,
        pltpu.SemaphoreType.DMA,
    ]
    if with_deg:
        out_types.append(jax.ShapeDtypeStruct((NC * N_NODES,), jnp.float32))
        scratch += [
            pltpu.VMEM((K,), jnp.float32),
            pltpu.VMEM((1008,), jnp.float32),
            pltpu.VMEM_SHARED((N_NODES,), jnp.float32),
            pltpu.SemaphoreType.DMA,
        ]

    mesh = plsc.VectorSubcoreMesh(
        core_axis_name="c", subcore_axis_name="s",
        num_cores=NC, num_subcores=NS)

    @functools.partial(pl.kernel, out_type=out_types, mesh=mesh,
                       scratch_types=scratch)
    def seg_sum(feats, src, dstr, *refs):
        if with_deg:
            (out_hbm, deg_hbm, sidx, didx, rows0, rows1, acc,
             gsem0, gsem1, ssem0, ssem1, ones, zdeg, dacc, degsem) = refs
        else:
            (out_hbm, sidx, didx, rows0, rows1, acc,
             gsem0, gsem1, ssem0, ssem1) = refs
        c = lax.axis_index("c")
        s = lax.axis_index("s")
        wid = s * NC + c
        rows = (rows0, rows1)
        gsem = (gsem0, gsem1)
        ssem = (ssem0, ssem1)

        pltpu.async_copy(src.at[pl.ds(wid * e_per_w, e_per_w)], sidx, gsem0)
        pltpu.async_copy(dstr.at[wid], didx, gsem1)

        def zrow_body(i, carry):
            for j in range(d // 16):
                rows0[i, pl.ds(j * 16, 16)] = jnp.zeros((16,), jnp.float32)
            return carry
        lax.fori_loop(0, K, zrow_body, 0)
        if with_deg:
            for j in range(K // 16):
                ones[pl.ds(j * 16, 16)] = jnp.ones((16,), jnp.float32)

            def zdeg_body(i, carry):
                zdeg[pl.ds(i * 16, 16)] = jnp.zeros((16,), jnp.float32)
                return carry
            lax.fori_loop(0, 1008 // 16, zdeg_body, 0)

        z_full = ROWS_PER_TILE // K
        z_rem = ROWS_PER_TILE - z_full * K
        for z in range(z_full):
            pltpu.async_copy(
                rows0, acc.at[pl.ds(s * ROWS_PER_TILE + z * K, K)], ssem0)
        if z_rem:
            pltpu.async_copy(
                rows0.at[pl.ds(0, z_rem)],
                acc.at[pl.ds(s * ROWS_PER_TILE + ROWS_PER_TILE - z_rem,
                             z_rem)], ssem1)
        @pl.when(s == 0)
        def _():
            pltpu.sync_copy(rows0.at[pl.ds(0, TAIL_ROWS)],
                            acc.at[pl.ds(TAIL_BASE, TAIL_ROWS)])
        if with_deg:
            @pl.when(s < N_NODES // DEG_CHUNK)
            def _():
                pltpu.sync_copy(zdeg.at[pl.ds(0, DEG_CHUNK)],
                                dacc.at[pl.ds(s * DEG_CHUNK, DEG_CHUNK)])
        pltpu.make_async_copy(
            src.at[pl.ds(0, e_per_w)], sidx, gsem0).wait()
        pltpu.make_async_copy(dstr.at[0], didx, gsem1).wait()
        for z in range(z_full):
            pltpu.make_async_copy(
                rows0, acc.at[pl.ds(0, K)], ssem0).wait()
        if z_rem:
            pltpu.make_async_copy(
                rows0.at[pl.ds(0, z_rem)], acc.at[pl.ds(0, z_rem)],
                ssem1).wait()
        plsc.subcore_barrier()

        def gather(k, b):
            pltpu.async_copy(
                feats.at[sidx.at[pl.ds(k * K, K)]], rows[b], gsem[b])

        def scatter(k, b):
            pltpu.async_copy(rows[b], acc.at[didx.at[k]], ssem[b], add=True)
            if with_deg:
                pltpu.async_copy(ones, dacc.at[didx.at[k]], degsem, add=True)

        def wait_scatter(b):
            pltpu.make_async_copy(rows[b], acc.at[didx.at[0]], ssem[b]).wait()

        def wait_gather(b):
            pltpu.make_async_copy(
                feats.at[sidx.at[pl.ds(0, K)]], rows[b], gsem[b]).wait()

        gather(0, 0)
        gather(1, 1)
        wait_gather(0)
        scatter(0, 0)
        wait_gather(1)
        scatter(1, 1)

        def pair(k2, carry):
            c0 = 2 * k2
            for b in range(2):
                wait_scatter(b)
                gather(c0 + b, b)
            for b in range(2):
                wait_gather(b)
                scatter(c0 + b, b)
            return carry
        lax.fori_loop(1, n_pairs, pair, 0)

        for t in range(tail_chunks):
            k = 2 * n_pairs + t
            wait_scatter(0)
            gather(k, 0)
            wait_gather(0)
            scatter(k, 0)

        for b in range(2):
            wait_scatter(b)
        if with_deg:
            def deg_drain(i, carry):
                pltpu.make_async_copy(
                    ones, dacc.at[didx.at[0]], degsem).wait()
                return carry
            lax.fori_loop(0, n_chunks, deg_drain, 0)
        plsc.subcore_barrier()

        pltpu.sync_copy(
            acc.at[pl.ds(s * ROWS_PER_TILE, ROWS_PER_TILE)],
            out_hbm.at[pl.ds(c * N_NODES + s * ROWS_PER_TILE, ROWS_PER_TILE)])
        @pl.when(s == 0)
        def _():
            pltpu.sync_copy(
                acc.at[pl.ds(TAIL_BASE, TAIL_ROWS)],
                out_hbm.at[pl.ds(c * N_NODES + TAIL_BASE, TAIL_ROWS)])
        if with_deg:
            @pl.when(s < N_NODES // DEG_CHUNK)
            def _():
                pltpu.sync_copy(dacc.at[pl.ds(s * DEG_CHUNK, DEG_CHUNK)],
                                zdeg.at[pl.ds(0, DEG_CHUNK)])
                pltpu.sync_copy(
                    zdeg.at[pl.ds(0, DEG_CHUNK)],
                    deg_hbm.at[pl.ds(c * N_NODES + s * DEG_CHUNK, DEG_CHUNK)])

    return seg_sum


_seg_sum_x = _make_seg_sum(IN_FEATS, with_deg=True)
_seg_sum_g = _make_seg_sum(IN_FEATS, with_deg=False)

_NB = 1000


def _dense_body(p0, p1, d0, d1, w1t, b1, w2t, g):
    deg = jnp.maximum(d0[...] + d1[...], 1.0)
    a = (p0[...] + p1[...]) / deg
    h = jnp.dot(a, w1t[...], preferred_element_type=jnp.float32) + b1[...]
    h = jnp.maximum(h, 0.0)
    g2 = jnp.dot(h, w2t[...], preferred_element_type=jnp.float32)
    g[...] = jnp.concatenate(
        [g2, jnp.zeros((_NB, IN_FEATS - NUM_CLASSES), jnp.float32)], axis=1)


def _combine_body(q0, q1, d0, d1, b2, o):
    deg = jnp.maximum(d0[...] + d1[...], 1.0)
    q = q0[:, :NUM_CLASSES] + q1[:, :NUM_CLASSES]
    o[...] = q / deg + b2[...]


_NBLK = N_NODES // _NB

_dense = pl.pallas_call(
    _dense_body,
    grid=(_NBLK,),
    in_specs=[
        pl.BlockSpec((_NB, IN_FEATS), lambda i: (i, 0)),
        pl.BlockSpec((_NB, IN_FEATS), lambda i: (i + _NBLK, 0)),
        pl.BlockSpec((_NB, 1), lambda i: (i, 0)),
        pl.BlockSpec((_NB, 1), lambda i: (i + _NBLK, 0)),
        pl.BlockSpec((IN_FEATS, H_FEATS), lambda i: (0, 0)),
        pl.BlockSpec((1, H_FEATS), lambda i: (0, 0)),
        pl.BlockSpec((H_FEATS, NUM_CLASSES), lambda i: (0, 0)),
    ],
    out_specs=pl.BlockSpec((_NB, IN_FEATS), lambda i: (i, 0)),
    out_shape=jax.ShapeDtypeStruct((N_NODES, IN_FEATS), jnp.float32),
)

_combine = pl.pallas_call(
    _combine_body,
    grid=(_NBLK,),
    in_specs=[
        pl.BlockSpec((_NB, IN_FEATS), lambda i: (i, 0)),
        pl.BlockSpec((_NB, IN_FEATS), lambda i: (i + _NBLK, 0)),
        pl.BlockSpec((_NB, 1), lambda i: (i, 0)),
        pl.BlockSpec((_NB, 1), lambda i: (i + _NBLK, 0)),
        pl.BlockSpec((1, NUM_CLASSES), lambda i: (0, 0)),
    ],
    out_specs=pl.BlockSpec((_NB, NUM_CLASSES), lambda i: (i, 0)),
    out_shape=jax.ShapeDtypeStruct((N_NODES, NUM_CLASSES), jnp.float32),
)


_N_CHUNKS = N_EDGES // NW // K


def kernel(x, edge_index, W1, b1, W2, b2):
    src = edge_index[0].astype(jnp.int32)
    dst = edge_index[1].astype(jnp.int32).reshape(NW, _N_CHUNKS, K)

    part1, degp = _seg_sum_x(x, src, dst)
    degc = degp.reshape(NC * N_NODES, 1)
    g = _dense(part1, part1, degc, degc,
               W1.T, b1.reshape(1, H_FEATS), W2.T)
    (part2,) = _seg_sum_g(g, src, dst)
    out = _combine(part2, part2, degc, degc,
                   b2.reshape(1, NUM_CLASSES))
    return out
